# baseline (speedup 1.0000x reference)
import functools
import math

import jax
import jax.numpy as jnp
from jax import lax
from jax.experimental import pallas as pl
from jax.experimental.pallas import tpu as pltpu
from jax.experimental.pallas import tpu_sc as plsc

N_ROWS = 16384
SIZE = 1000
EPS = 0.1 / (SIZE - 2)
CONF = 0.9
C_ENT = CONF * math.log(CONF) + (SIZE - 2) * EPS * math.log(EPS)
CME = CONF - EPS

NC = 2
NS = 16
L = 16
NW = NC * NS
ROWS_PER_W = N_ROWS // NW
GCHUNK = 128
NCHUNK = ROWS_PER_W // GCHUNK

KSTREAM = 4
TBS = 512
TBT = KSTREAM * TBS
TG = N_ROWS // TBT


def _sc_gather_body(x_hbm, tgt_hbm, out_hbm, t_v, idx_v, g_v, acc_v, sem):
    wid = lax.axis_index("s") * NC + lax.axis_index("c")
    base = wid * ROWS_PER_W
    pltpu.sync_copy(tgt_hbm.at[pl.ds(base, ROWS_PER_W)], t_v)
    iota = lax.iota(jnp.int32, L)
    for k in range(ROWS_PER_W // L):
        tv = t_v[pl.ds(k * L, L)]
        rows = base + k * L + iota
        idx_v[k // (GCHUNK // L), pl.ds((k % (GCHUNK // L)) * L, L)] = (
            rows * SIZE + tv
        )
    for j in range(NCHUNK):
        pltpu.async_copy(x_hbm.at[idx_v.at[j]], g_v.at[j], sem).wait()
    acc = jnp.zeros((L,), jnp.float32)
    for k in range(ROWS_PER_W // L):
        tv = t_v[pl.ds(k * L, L)]
        gv = g_v[k // (GCHUNK // L), pl.ds((k % (GCHUNK // L)) * L, L)]
        acc = acc + jnp.where(tv != 0, gv, jnp.float32(0.0))
    acc_v[...] = acc
    pltpu.sync_copy(acc_v, out_hbm.at[wid])


@functools.lru_cache(maxsize=None)
def _make_sc_gather():
    return functools.partial(
        pl.kernel,
        mesh=plsc.VectorSubcoreMesh(core_axis_name="c", subcore_axis_name="s"),
        out_type=jax.ShapeDtypeStruct((NW, L), jnp.float32),
        scratch_types=[
            pltpu.VMEM((ROWS_PER_W,), jnp.int32),
            pltpu.VMEM((NCHUNK, GCHUNK), jnp.int32),
            pltpu.VMEM((NCHUNK, GCHUNK), jnp.float32),
            pltpu.VMEM((L,), jnp.float32),
            pltpu.SemaphoreType.DMA,
        ],
    )(_sc_gather_body)


def _tc_body(*refs):
    x_refs = refs[:KSTREAM]
    t_refs = refs[KSTREAM:2 * KSTREAM]
    o_ref = refs[2 * KSTREAM]
    i = pl.program_id(0)
    s = jnp.float32(0.0)
    for k, x_ref in enumerate(x_refs):
        xb = x_ref[...]
        tk = t_refs[k][0]
        nonpad = tk != 0
        rs = jnp.sum(xb, axis=1, keepdims=True)
        x0 = xb[:, 0:1]
        contrib = jnp.where(
            nonpad,
            jnp.float32(C_ENT) - jnp.float32(EPS) * (rs - x0),
            jnp.float32(0.0),
        )
        s = s + jnp.sum(contrib)

    @pl.when(i == 0)
    def _init():
        o_ref[0, 0] = jnp.float32(0.0)

    o_ref[0, 0] += s


def kernel(x, target):
    t32 = target.astype(jnp.int32)
    gpart = _make_sc_gather()(x.reshape(-1), t32)
    t3 = t32.reshape(N_ROWS // TBS, TBS, 1)
    x_specs = [
        pl.BlockSpec((TBS, SIZE), lambda i, k=k: (i * KSTREAM + k, 0))
        for k in range(KSTREAM)
    ]
    t_specs = [
        pl.BlockSpec((1, TBS, 1), lambda i, k=k: (i * KSTREAM + k, 0, 0))
        for k in range(KSTREAM)
    ]
    out = pl.pallas_call(
        _tc_body,
        grid=(TG,),
        in_specs=x_specs + t_specs,
        out_specs=pl.BlockSpec(
            (1, 1), lambda i: (0, 0), memory_space=pltpu.SMEM
        ),
        out_shape=jax.ShapeDtypeStruct((1, 1), jnp.float32),
        compiler_params=pltpu.CompilerParams(
            dimension_semantics=("arbitrary",),
        ),
    )(*([x] * KSTREAM), *([t3] * KSTREAM))
    return out[0, 0] - jnp.float32(CME) * jnp.sum(gpart)


# speedup vs baseline: 1.9983x; 1.9983x over previous
import functools
import math

import jax
import jax.numpy as jnp
from jax import lax
from jax.experimental import pallas as pl
from jax.experimental.pallas import tpu as pltpu
from jax.experimental.pallas import tpu_sc as plsc

N_ROWS = 16384
SIZE = 1000
EPS = 0.1 / (SIZE - 2)
CONF = 0.9
C_ENT = CONF * math.log(CONF) + (SIZE - 2) * EPS * math.log(EPS)
CME = CONF - EPS

NC = 2
NS = 16
L = 16
NW = NC * NS
ROWS_PER_W = N_ROWS // NW
GCHUNK = 128
NCHUNK = ROWS_PER_W // GCHUNK

KSTREAM = 8
TBS = 256
TBT = KSTREAM * TBS
TG = N_ROWS // TBT


def _sc_gather_body(x_hbm, tgt_hbm, out_hbm, t_v, idx_v, g_v, acc_v, sem):
    wid = lax.axis_index("s") * NC + lax.axis_index("c")
    base = wid * ROWS_PER_W
    pltpu.sync_copy(tgt_hbm.at[pl.ds(base, ROWS_PER_W)], t_v)
    iota = lax.iota(jnp.int32, L)
    for k in range(ROWS_PER_W // L):
        tv = t_v[pl.ds(k * L, L)]
        rows = base + k * L + iota
        idx_v[k // (GCHUNK // L), pl.ds((k % (GCHUNK // L)) * L, L)] = (
            rows * SIZE + tv
        )
    for j in range(NCHUNK):
        pltpu.async_copy(x_hbm.at[idx_v.at[j]], g_v.at[j], sem).wait()
    acc = jnp.zeros((L,), jnp.float32)
    for k in range(ROWS_PER_W // L):
        tv = t_v[pl.ds(k * L, L)]
        gv = g_v[k // (GCHUNK // L), pl.ds((k % (GCHUNK // L)) * L, L)]
        acc = acc + jnp.where(tv != 0, gv, jnp.float32(0.0))
    acc_v[...] = acc
    pltpu.sync_copy(acc_v, out_hbm.at[wid])


@functools.lru_cache(maxsize=None)
def _make_sc_gather():
    return functools.partial(
        pl.kernel,
        mesh=plsc.VectorSubcoreMesh(core_axis_name="c", subcore_axis_name="s"),
        out_type=jax.ShapeDtypeStruct((NW, L), jnp.float32),
        scratch_types=[
            pltpu.VMEM((ROWS_PER_W,), jnp.int32),
            pltpu.VMEM((NCHUNK, GCHUNK), jnp.int32),
            pltpu.VMEM((NCHUNK, GCHUNK), jnp.float32),
            pltpu.VMEM((L,), jnp.float32),
            pltpu.SemaphoreType.DMA,
        ],
    )(_sc_gather_body)


def _tc_body(*refs):
    x_refs = refs[:KSTREAM]
    t_refs = refs[KSTREAM:2 * KSTREAM]
    o_ref = refs[2 * KSTREAM]
    i = pl.program_id(0)
    s = jnp.float32(0.0)
    for k, x_ref in enumerate(x_refs):
        xb = x_ref[...]
        tk = t_refs[k][0]
        nonpad = tk != 0
        rs = jnp.sum(xb, axis=1, keepdims=True)
        x0 = xb[:, 0:1]
        cols = jax.lax.broadcasted_iota(jnp.int32, (TBS, SIZE), 1)
        gk = jnp.sum(
            jnp.where(cols == tk, xb, jnp.float32(0.0)),
            axis=1, keepdims=True,
        )
        contrib = jnp.where(
            nonpad,
            jnp.float32(C_ENT)
            - jnp.float32(EPS) * (rs - x0)
            - jnp.float32(CME) * gk,
            jnp.float32(0.0),
        )
        s = s + jnp.sum(contrib)

    @pl.when(i == 0)
    def _init():
        o_ref[0, 0] = jnp.float32(0.0)

    o_ref[0, 0] += s


def kernel(x, target):
    t32 = target.astype(jnp.int32)
    t3 = t32.reshape(N_ROWS // TBS, TBS, 1)
    x_specs = [
        pl.BlockSpec((TBS, SIZE), lambda i, k=k: (i * KSTREAM + k, 0))
        for k in range(KSTREAM)
    ]
    t_specs = [
        pl.BlockSpec((1, TBS, 1), lambda i, k=k: (i * KSTREAM + k, 0, 0))
        for k in range(KSTREAM)
    ]
    out = pl.pallas_call(
        _tc_body,
        grid=(TG,),
        in_specs=x_specs + t_specs,
        out_specs=pl.BlockSpec(
            (1, 1), lambda i: (0, 0), memory_space=pltpu.SMEM
        ),
        out_shape=jax.ShapeDtypeStruct((1, 1), jnp.float32),
        compiler_params=pltpu.CompilerParams(
            dimension_semantics=("arbitrary",),
        ),
    )(*([x] * KSTREAM), *([t3] * KSTREAM))
    return out[0, 0]


# speedup vs baseline: 2.0254x; 1.0136x over previous
import functools
import math

import jax
import jax.numpy as jnp
from jax import lax
from jax.experimental import pallas as pl
from jax.experimental.pallas import tpu as pltpu
from jax.experimental.pallas import tpu_sc as plsc

N_ROWS = 16384
SIZE = 1000
EPS = 0.1 / (SIZE - 2)
CONF = 0.9
C_ENT = CONF * math.log(CONF) + (SIZE - 2) * EPS * math.log(EPS)
CME = CONF - EPS

NC = 2
NS = 16
L = 16
NW = NC * NS
ROWS_PER_W = N_ROWS // NW
GCHUNK = 128
NCHUNK = ROWS_PER_W // GCHUNK

KSTREAM = 2
TBS = 1024
TBT = KSTREAM * TBS
TG = N_ROWS // TBT


def _sc_gather_body(x_hbm, tgt_hbm, out_hbm, t_v, idx_v, g_v, acc_v, sem):
    wid = lax.axis_index("s") * NC + lax.axis_index("c")
    base = wid * ROWS_PER_W
    pltpu.sync_copy(tgt_hbm.at[pl.ds(base, ROWS_PER_W)], t_v)
    iota = lax.iota(jnp.int32, L)
    for k in range(ROWS_PER_W // L):
        tv = t_v[pl.ds(k * L, L)]
        rows = base + k * L + iota
        idx_v[k // (GCHUNK // L), pl.ds((k % (GCHUNK // L)) * L, L)] = (
            rows * SIZE + tv
        )
    for j in range(NCHUNK):
        pltpu.async_copy(x_hbm.at[idx_v.at[j]], g_v.at[j], sem).wait()
    acc = jnp.zeros((L,), jnp.float32)
    for k in range(ROWS_PER_W // L):
        tv = t_v[pl.ds(k * L, L)]
        gv = g_v[k // (GCHUNK // L), pl.ds((k % (GCHUNK // L)) * L, L)]
        acc = acc + jnp.where(tv != 0, gv, jnp.float32(0.0))
    acc_v[...] = acc
    pltpu.sync_copy(acc_v, out_hbm.at[wid])


@functools.lru_cache(maxsize=None)
def _make_sc_gather():
    return functools.partial(
        pl.kernel,
        mesh=plsc.VectorSubcoreMesh(core_axis_name="c", subcore_axis_name="s"),
        out_type=jax.ShapeDtypeStruct((NW, L), jnp.float32),
        scratch_types=[
            pltpu.VMEM((ROWS_PER_W,), jnp.int32),
            pltpu.VMEM((NCHUNK, GCHUNK), jnp.int32),
            pltpu.VMEM((NCHUNK, GCHUNK), jnp.float32),
            pltpu.VMEM((L,), jnp.float32),
            pltpu.SemaphoreType.DMA,
        ],
    )(_sc_gather_body)


def _tc_body(*refs):
    x_refs = refs[:KSTREAM]
    t_refs = refs[KSTREAM:2 * KSTREAM]
    o_ref = refs[2 * KSTREAM]
    i = pl.program_id(0)
    s = jnp.float32(0.0)
    for k, x_ref in enumerate(x_refs):
        xb = x_ref[...]
        tk = t_refs[k][0]
        nonpad = tk != 0
        rs = jnp.sum(xb, axis=1, keepdims=True)
        x0 = xb[:, 0:1]
        cols = jax.lax.broadcasted_iota(jnp.int32, (TBS, SIZE), 1)
        gk = jnp.sum(
            jnp.where(cols == tk, xb, jnp.float32(0.0)),
            axis=1, keepdims=True,
        )
        contrib = jnp.where(
            nonpad,
            jnp.float32(C_ENT)
            - jnp.float32(EPS) * (rs - x0)
            - jnp.float32(CME) * gk,
            jnp.float32(0.0),
        )
        s = s + jnp.sum(contrib)

    @pl.when(i == 0)
    def _init():
        o_ref[0, 0] = jnp.float32(0.0)

    o_ref[0, 0] += s


def kernel(x, target):
    t32 = target.astype(jnp.int32)
    t3 = t32.reshape(N_ROWS // TBS, TBS, 1)
    x_specs = [
        pl.BlockSpec((TBS, SIZE), lambda i, k=k: (i * KSTREAM + k, 0))
        for k in range(KSTREAM)
    ]
    t_specs = [
        pl.BlockSpec((1, TBS, 1), lambda i, k=k: (i * KSTREAM + k, 0, 0))
        for k in range(KSTREAM)
    ]
    out = pl.pallas_call(
        _tc_body,
        grid=(TG,),
        in_specs=x_specs + t_specs,
        out_specs=pl.BlockSpec(
            (1, 1), lambda i: (0, 0), memory_space=pltpu.SMEM
        ),
        out_shape=jax.ShapeDtypeStruct((1, 1), jnp.float32),
        compiler_params=pltpu.CompilerParams(
            dimension_semantics=("arbitrary",),
        ),
    )(*([x] * KSTREAM), *([t3] * KSTREAM))
    return out[0, 0]
